# baseline (speedup 1.0000x reference)
import functools

import jax
import jax.numpy as jnp
from jax import lax
from jax.experimental import pallas as pl
from jax.experimental.pallas import tpu as pltpu
from jax.experimental.pallas import tpu_sc as plsc

_HIDDEN = 4096
_EXPERTS = 64
_TOPK = 8
_BLOCK_T = 1024
_N_TOKENS = 16384
_NW = 32
_ROWS_PER_W = _N_TOKENS // _NW
_LANES = 16


def _softmax_block_kernel(hs_ref, w_ref, p_ref):
    logits = jax.lax.dot_general(
        hs_ref[...], w_ref[...], (((1,), (1,)), ((), ())),
        preferred_element_type=jnp.float32)
    e = jnp.exp(logits)
    p_ref[...] = e / jnp.sum(e, axis=-1, keepdims=True)


def _tc_softmax(hs, weight):
    return pl.pallas_call(
        _softmax_block_kernel,
        grid=(_N_TOKENS // _BLOCK_T,),
        in_specs=[
            pl.BlockSpec((_BLOCK_T, _HIDDEN), lambda i: (i, 0)),
            pl.BlockSpec((_EXPERTS, _HIDDEN), lambda i: (0, 0)),
        ],
        out_specs=pl.BlockSpec((_BLOCK_T, _EXPERTS), lambda i: (i, 0)),
        out_shape=jax.ShapeDtypeStruct((_N_TOKENS, _EXPERTS), jnp.float32),
        compiler_params=pltpu.CompilerParams(
            dimension_semantics=("parallel",)),
    )(hs, weight)


def _sc_topk_body(p_hbm, scores_hbm, idx_hbm, p_v, s_v, i_v):
    wid = lax.axis_index("s") * 2 + lax.axis_index("c")
    base = wid * _ROWS_PER_W
    pltpu.sync_copy(p_hbm.at[pl.ds(base, _ROWS_PER_W)], p_v)

    iota16 = lax.iota(jnp.int32, _LANES)
    mask8 = iota16 < _TOPK

    def merge(a, b):
        m = jnp.maximum(a, lax.rev(b, (0,)))
        s, _ = plsc.sort_key_val(m, m, descending=True)
        return s

    def row(i, carry):
        chunks = []
        for c in range(_EXPERTS // _LANES):
            pc = p_v[i, pl.ds(c * _LANES, _LANES)]
            b = lax.bitcast_convert_type(pc, jnp.int32)
            k = (b & ~0x3F) | (0x3F - (iota16 + c * _LANES))
            kf = lax.bitcast_convert_type(k, jnp.float32)
            s, _ = plsc.sort_key_val(kf, kf, descending=True)
            chunks.append(s)
        h = merge(merge(chunks[0], chunks[1]), merge(chunks[2], chunks[3]))
        ki = lax.bitcast_convert_type(h, jnp.int32)
        idx = 0x3F - (ki & 0x3F)
        val = lax.bitcast_convert_type((ki & ~0x3F) | 0x20, jnp.float32)
        denom = jnp.sum(jnp.where(mask8, val, 0.0))
        s_v[i, :] = val / denom
        i_v[i, :] = idx
        return carry

    lax.fori_loop(0, _ROWS_PER_W, row, 0)

    pltpu.sync_copy(s_v, scores_hbm.at[pl.ds(base, _ROWS_PER_W)])
    pltpu.sync_copy(i_v, idx_hbm.at[pl.ds(base, _ROWS_PER_W)])


_sc_topk = functools.partial(
    pl.kernel,
    mesh=plsc.VectorSubcoreMesh(core_axis_name="c", subcore_axis_name="s"),
    out_type=[
        jax.ShapeDtypeStruct((_N_TOKENS, _LANES), jnp.float32),
        jax.ShapeDtypeStruct((_N_TOKENS, _LANES), jnp.int32),
    ],
    scratch_types=[
        pltpu.VMEM((_ROWS_PER_W, _EXPERTS), jnp.float32),
        pltpu.VMEM((_ROWS_PER_W, _LANES), jnp.float32),
        pltpu.VMEM((_ROWS_PER_W, _LANES), jnp.int32),
    ],
    compiler_params=pltpu.CompilerParams(
        needs_layout_passes=False, use_tc_tiling_on_sc=False),
)(_sc_topk_body)


def kernel(hidden_states, weight):
    hs = hidden_states.reshape(-1, _HIDDEN)
    p = _tc_softmax(hs, weight)
    scores16, idx16 = _sc_topk(p)
    return (p, scores16[:, :_TOPK], idx16[:, :_TOPK])


# speedup vs baseline: 1.3887x; 1.3887x over previous
import jax
import jax.numpy as jnp
from jax.experimental import pallas as pl
from jax.experimental.pallas import tpu as pltpu

_HIDDEN = 4096
_EXPERTS = 64
_TOPK = 8
_BLOCK_T = 512


def _router_block_kernel(hs_ref, w_ref, logits_ref, scores_ref, idx_ref):
    logits = jax.lax.dot_general(
        hs_ref[...], w_ref[...], (((1,), (1,)), ((), ())),
        preferred_element_type=jnp.float32)

    e = jnp.exp(logits)
    p = e / jnp.sum(e, axis=-1, keepdims=True)
    logits_ref[...] = p

    iota = jax.lax.broadcasted_iota(jnp.int32, e.shape, 1)
    ebits = jax.lax.bitcast_convert_type(e, jnp.int32)
    key = jax.lax.bitcast_convert_type(
        (ebits & ~0x3F) | (0x3F - iota), jnp.float32)
    vals = []
    idxs = []
    for _ in range(_TOPK):
        mk = jnp.max(key, axis=-1, keepdims=True)
        key = jnp.where(key == mk, -1.0, key)
        mbits = jax.lax.bitcast_convert_type(mk, jnp.int32)
        idxs.append(0x3F - (mbits & 0x3F))
        vals.append(jax.lax.bitcast_convert_type(
            (mbits & ~0x3F) | 0x20, jnp.float32))
    topv = jnp.concatenate(vals, axis=-1)
    topi = jnp.concatenate(idxs, axis=-1)
    scores_ref[...] = topv / jnp.sum(topv, axis=-1, keepdims=True)
    idx_ref[...] = topi


def kernel(hidden_states, weight):
    hs = hidden_states.reshape(-1, _HIDDEN)
    n_tokens = hs.shape[0]
    grid = (n_tokens // _BLOCK_T,)

    logits, scores, indices = pl.pallas_call(
        _router_block_kernel,
        grid=grid,
        in_specs=[
            pl.BlockSpec((_BLOCK_T, _HIDDEN), lambda i: (i, 0)),
            pl.BlockSpec((_EXPERTS, _HIDDEN), lambda i: (0, 0)),
        ],
        out_specs=[
            pl.BlockSpec((_BLOCK_T, _EXPERTS), lambda i: (i, 0)),
            pl.BlockSpec((_BLOCK_T, _TOPK), lambda i: (i, 0)),
            pl.BlockSpec((_BLOCK_T, _TOPK), lambda i: (i, 0)),
        ],
        out_shape=[
            jax.ShapeDtypeStruct((n_tokens, _EXPERTS), jnp.float32),
            jax.ShapeDtypeStruct((n_tokens, _TOPK), jnp.float32),
            jax.ShapeDtypeStruct((n_tokens, _TOPK), jnp.int32),
        ],
        compiler_params=pltpu.CompilerParams(
            dimension_semantics=("parallel",)),
    )(hs, weight)
    return (logits, scores, indices)


# speedup vs baseline: 1.4820x; 1.0672x over previous
import jax
import jax.numpy as jnp
from jax.experimental import pallas as pl
from jax.experimental.pallas import tpu as pltpu

_HIDDEN = 4096
_EXPERTS = 64
_TOPK = 8
_BLOCK_T = 1024


def _router_block_kernel(hs_ref, w_ref, logits_ref, scores_ref, idx_ref):
    logits = jax.lax.dot_general(
        hs_ref[...], w_ref[...], (((1,), (1,)), ((), ())),
        preferred_element_type=jnp.float32)

    e = jnp.exp(logits)
    p = e / jnp.sum(e, axis=-1, keepdims=True)
    logits_ref[...] = p

    iota = jax.lax.broadcasted_iota(jnp.int32, e.shape, 1)
    ebits = jax.lax.bitcast_convert_type(e, jnp.int32)
    key = jax.lax.bitcast_convert_type(
        (ebits & ~0x3F) | (0x3F - iota), jnp.float32)
    vals = []
    idxs = []
    for _ in range(_TOPK):
        mk = jnp.max(key, axis=-1, keepdims=True)
        key = jnp.where(key == mk, -1.0, key)
        mbits = jax.lax.bitcast_convert_type(mk, jnp.int32)
        idxs.append(0x3F - (mbits & 0x3F))
        vals.append(jax.lax.bitcast_convert_type(
            (mbits & ~0x3F) | 0x20, jnp.float32))
    topv = jnp.concatenate(vals, axis=-1)
    topi = jnp.concatenate(idxs, axis=-1)
    scores_ref[...] = topv / jnp.sum(topv, axis=-1, keepdims=True)
    idx_ref[...] = topi


def kernel(hidden_states, weight):
    hs = hidden_states.reshape(-1, _HIDDEN)
    n_tokens = hs.shape[0]
    grid = (n_tokens // _BLOCK_T,)

    logits, scores, indices = pl.pallas_call(
        _router_block_kernel,
        grid=grid,
        in_specs=[
            pl.BlockSpec((_BLOCK_T, _HIDDEN), lambda i: (i, 0)),
            pl.BlockSpec((_EXPERTS, _HIDDEN), lambda i: (0, 0)),
        ],
        out_specs=[
            pl.BlockSpec((_BLOCK_T, _EXPERTS), lambda i: (i, 0)),
            pl.BlockSpec((_BLOCK_T, _TOPK), lambda i: (i, 0)),
            pl.BlockSpec((_BLOCK_T, _TOPK), lambda i: (i, 0)),
        ],
        out_shape=[
            jax.ShapeDtypeStruct((n_tokens, _EXPERTS), jnp.float32),
            jax.ShapeDtypeStruct((n_tokens, _TOPK), jnp.float32),
            jax.ShapeDtypeStruct((n_tokens, _TOPK), jnp.int32),
        ],
        compiler_params=pltpu.CompilerParams(
            dimension_semantics=("parallel",)),
    )(hs, weight)
    return (logits, scores, indices)
